# baseline (speedup 1.0000x reference)
import functools
import math

import jax
import jax.numpy as jnp
from jax.experimental import pallas as pl
from jax.experimental.pallas import tpu as pltpu

_BLOCK_ROWS = 1024
_NSLOTS = 3


def _compute_block(x_blk, g, b, w, eps):
    f = x_blk.shape[-1]
    s1 = jnp.sum(x_blk, axis=-1, keepdims=True)
    s2 = jnp.sum(x_blk * x_blk, axis=-1, keepdims=True)
    mean = s1 * jnp.float32(1.0 / f)
    var = (s2 - mean * s1) * jnp.float32(1.0 / (f - 1))
    inv = pl.reciprocal(jnp.sqrt(var) + jnp.float32(eps), approx=False)
    h = (x_blk - mean) * inv * g + b
    y = jnp.dot(h.astype(jnp.bfloat16), w, preferred_element_type=jnp.float32)
    return x_blk + y


def _pipelined_kernel(x_hbm, g_ref, b_ref, w_hbm, o_hbm,
                      x_buf, o_buf, w_vmem, in_sem, out_sem, w_sem,
                      *, block: int, n_steps: int, eps: float):
    def dma_in(slot, step):
        pltpu.make_async_copy(x_hbm.at[pl.ds(step * block, block)],
                              x_buf.at[slot], in_sem.at[slot]).start()

    def wait_in(slot):
        pltpu.make_async_copy(x_hbm.at[pl.ds(0, block)],
                              x_buf.at[slot], in_sem.at[slot]).wait()

    def dma_out(slot, step):
        pltpu.make_async_copy(o_buf.at[slot],
                              o_hbm.at[pl.ds(step * block, block)],
                              out_sem.at[slot]).start()

    def wait_out(slot):
        pltpu.make_async_copy(o_buf.at[slot], o_hbm.at[pl.ds(0, block)],
                              out_sem.at[slot]).wait()

    g = g_ref[...]
    b = b_ref[...]

    pltpu.make_async_copy(w_hbm, w_vmem, w_sem).start()
    dma_in(0, 0)
    if n_steps > 1:
        dma_in(1 % _NSLOTS, 1)
    pltpu.make_async_copy(w_hbm, w_vmem, w_sem).wait()
    w = w_vmem[...]

    def body(step, _):
        cur = jax.lax.rem(step, _NSLOTS)

        @pl.when(step + 2 < n_steps)
        def _():
            dma_in(jax.lax.rem(step + 2, _NSLOTS), step + 2)

        wait_in(cur)

        @pl.when(step >= _NSLOTS)
        def _():
            wait_out(cur)

        o_buf[cur] = _compute_block(x_buf[cur], g, b, w, eps)
        dma_out(cur, step)
        return ()

    jax.lax.fori_loop(0, n_steps, body, ())
    for k in range(min(_NSLOTS, n_steps)):
        wait_out((n_steps - 1 - k) % _NSLOTS)


def kernel(x, a_2, b_2, w, eps: float = 1e-6):
    orig_shape = x.shape
    features = orig_shape[-1]
    rows = math.prod(orig_shape[:-1])
    x2 = x.reshape(rows, features)
    g2 = a_2.astype(jnp.float32).reshape(1, features)
    b2 = b_2.astype(jnp.float32).reshape(1, features)
    w_bf16 = w.astype(jnp.bfloat16)

    block = _BLOCK_ROWS
    while rows % block:
        block //= 2
    n_steps = rows // block

    out = pl.pallas_call(
        functools.partial(_pipelined_kernel, block=block, n_steps=n_steps,
                          eps=eps),
        out_shape=jax.ShapeDtypeStruct((rows, features), x.dtype),
        in_specs=[
            pl.BlockSpec(memory_space=pl.ANY),
            pl.BlockSpec(memory_space=pltpu.VMEM),
            pl.BlockSpec(memory_space=pltpu.VMEM),
            pl.BlockSpec(memory_space=pl.ANY),
        ],
        out_specs=pl.BlockSpec(memory_space=pl.ANY),
        scratch_shapes=[
            pltpu.VMEM((_NSLOTS, block, features), jnp.float32),
            pltpu.VMEM((_NSLOTS, block, features), jnp.float32),
            pltpu.VMEM((features, features), jnp.bfloat16),
            pltpu.SemaphoreType.DMA((_NSLOTS,)),
            pltpu.SemaphoreType.DMA((_NSLOTS,)),
            pltpu.SemaphoreType.DMA,
        ],
        compiler_params=pltpu.CompilerParams(
            vmem_limit_bytes=48 * 1024 * 1024,
        ),
    )(x2, g2, b2, w_bf16)

    return out.reshape(orig_shape)


# speedup vs baseline: 1.0497x; 1.0497x over previous
import functools
import math

import jax
import jax.numpy as jnp
from jax.experimental import pallas as pl
from jax.experimental.pallas import tpu as pltpu

_BLOCK_ROWS = 1024
_NSLOTS = 3


def _compute_block(x_blk, g, b, w, eps):
    f = x_blk.shape[-1]
    s1 = jnp.sum(x_blk, axis=-1, keepdims=True)
    s2 = jnp.sum(x_blk * x_blk, axis=-1, keepdims=True)
    mean = s1 * jnp.float32(1.0 / f)
    var = (s2 - mean * s1) * jnp.float32(1.0 / (f - 1))
    inv = pl.reciprocal(jnp.sqrt(var) + jnp.float32(eps), approx=False)
    h = (x_blk - mean) * inv * g + b
    y = jnp.dot(h.astype(jnp.bfloat16), w, preferred_element_type=jnp.float32)
    return x_blk + y


def _pipelined_kernel(x_hbm, g_ref, b_ref, w_ref, o_hbm,
                      x_buf, o_buf, in_sem, out_sem,
                      *, block: int, n_steps: int, eps: float):
    def dma_in(slot, step):
        pltpu.make_async_copy(x_hbm.at[pl.ds(step * block, block)],
                              x_buf.at[slot], in_sem.at[slot]).start()

    def wait_in(slot):
        pltpu.make_async_copy(x_hbm.at[pl.ds(0, block)],
                              x_buf.at[slot], in_sem.at[slot]).wait()

    hr = block // 2

    def dma_out(slot, step, half):
        pltpu.make_async_copy(o_buf.at[slot, pl.ds(half * hr, hr)],
                              o_hbm.at[pl.ds(step * block + half * hr, hr)],
                              out_sem.at[slot, half]).start()

    def wait_out(slot, half):
        pltpu.make_async_copy(o_buf.at[slot, pl.ds(0, hr)],
                              o_hbm.at[pl.ds(0, hr)],
                              out_sem.at[slot, half]).wait()

    g = g_ref[...]
    b = b_ref[...]
    w = w_ref[...]

    dma_in(0, 0)
    if n_steps > 1:
        dma_in(1 % _NSLOTS, 1)

    def body(step, _):
        cur = jax.lax.rem(step, _NSLOTS)

        @pl.when(step + 2 < n_steps)
        def _():
            dma_in(jax.lax.rem(step + 2, _NSLOTS), step + 2)

        wait_in(cur)

        @pl.when(step >= _NSLOTS)
        def _():
            wait_out(cur, 0)
            wait_out(cur, 1)

        o_buf[cur, pl.ds(0, hr)] = _compute_block(
            x_buf[cur, pl.ds(0, hr)], g, b, w, eps)
        dma_out(cur, step, 0)
        o_buf[cur, pl.ds(hr, hr)] = _compute_block(
            x_buf[cur, pl.ds(hr, hr)], g, b, w, eps)
        dma_out(cur, step, 1)
        return ()

    jax.lax.fori_loop(0, n_steps, body, ())
    for k in range(min(_NSLOTS, n_steps)):
        wait_out((n_steps - 1 - k) % _NSLOTS, 0)
        wait_out((n_steps - 1 - k) % _NSLOTS, 1)


def kernel(x, a_2, b_2, w, eps: float = 1e-6):
    orig_shape = x.shape
    features = orig_shape[-1]
    rows = math.prod(orig_shape[:-1])
    x2 = x.reshape(rows, features)
    g2 = a_2.astype(jnp.float32).reshape(1, features)
    b2 = b_2.astype(jnp.float32).reshape(1, features)
    w_bf16 = w.astype(jnp.bfloat16)

    block = _BLOCK_ROWS
    while rows % block:
        block //= 2
    n_steps = rows // block

    out = pl.pallas_call(
        functools.partial(_pipelined_kernel, block=block, n_steps=n_steps,
                          eps=eps),
        out_shape=jax.ShapeDtypeStruct((rows, features), x.dtype),
        in_specs=[
            pl.BlockSpec(memory_space=pl.ANY),
            pl.BlockSpec(memory_space=pltpu.VMEM),
            pl.BlockSpec(memory_space=pltpu.VMEM),
            pl.BlockSpec(memory_space=pltpu.VMEM),
        ],
        out_specs=pl.BlockSpec(memory_space=pl.ANY),
        scratch_shapes=[
            pltpu.VMEM((_NSLOTS, block, features), jnp.float32),
            pltpu.VMEM((_NSLOTS, block, features), jnp.float32),
            pltpu.SemaphoreType.DMA((_NSLOTS,)),
            pltpu.SemaphoreType.DMA((_NSLOTS, 2)),
        ],
        compiler_params=pltpu.CompilerParams(
            vmem_limit_bytes=48 * 1024 * 1024,
        ),
    )(x2, g2, b2, w_bf16)

    return out.reshape(orig_shape)


# speedup vs baseline: 1.1099x; 1.0574x over previous
import functools
import math

import jax
import jax.numpy as jnp
from jax.experimental import pallas as pl
from jax.experimental.pallas import tpu as pltpu

_BLOCK_ROWS = 1024
_NSLOTS = 3


def _compute_block(x_blk, g, b, w, eps):
    f = x_blk.shape[-1]
    s1 = jnp.sum(x_blk, axis=-1, keepdims=True)
    s2 = jnp.sum(x_blk * x_blk, axis=-1, keepdims=True)
    mean = s1 * jnp.float32(1.0 / f)
    var = (s2 - mean * s1) * jnp.float32(1.0 / (f - 1))
    inv = pl.reciprocal(jnp.sqrt(var) + jnp.float32(eps), approx=False)
    h = (x_blk - mean) * inv * g + b
    y = jnp.dot(h.astype(jnp.bfloat16), w, preferred_element_type=jnp.float32)
    return x_blk + y


def _pipelined_kernel(x_hbm, g_ref, b_ref, w_ref, o_hbm,
                      x_buf, o_buf, in_sem, out_sem,
                      *, block: int, n_steps: int, eps: float):
    def dma_in(slot, step):
        pltpu.make_async_copy(x_hbm.at[pl.ds(step * block, block)],
                              x_buf.at[slot], in_sem.at[slot]).start()

    def wait_in(slot):
        pltpu.make_async_copy(x_hbm.at[pl.ds(0, block)],
                              x_buf.at[slot], in_sem.at[slot]).wait()

    hr = block // 4

    def dma_out(slot, step, half):
        pltpu.make_async_copy(o_buf.at[slot, pl.ds(half * hr, hr)],
                              o_hbm.at[pl.ds(step * block + half * hr, hr)],
                              out_sem.at[slot, half]).start()

    def wait_out(slot, half):
        pltpu.make_async_copy(o_buf.at[slot, pl.ds(0, hr)],
                              o_hbm.at[pl.ds(0, hr)],
                              out_sem.at[slot, half]).wait()

    g = g_ref[...]
    b = b_ref[...]
    w = w_ref[...]

    dma_in(0, 0)
    if n_steps > 1:
        dma_in(1 % _NSLOTS, 1)

    def body(step, _):
        cur = jax.lax.rem(step, _NSLOTS)

        @pl.when(step + 2 < n_steps)
        def _():
            dma_in(jax.lax.rem(step + 2, _NSLOTS), step + 2)

        wait_in(cur)

        @pl.when(step >= _NSLOTS)
        def _():
            for q in range(4):
                wait_out(cur, q)

        for q in range(4):
            o_buf[cur, pl.ds(q * hr, hr)] = _compute_block(
                x_buf[cur, pl.ds(q * hr, hr)], g, b, w, eps)
            dma_out(cur, step, q)
        return ()

    jax.lax.fori_loop(0, n_steps, body, ())
    for k in range(min(_NSLOTS, n_steps)):
        for q in range(4):
            wait_out((n_steps - 1 - k) % _NSLOTS, q)


def kernel(x, a_2, b_2, w, eps: float = 1e-6):
    orig_shape = x.shape
    features = orig_shape[-1]
    rows = math.prod(orig_shape[:-1])
    x2 = x.reshape(rows, features)
    g2 = a_2.astype(jnp.float32).reshape(1, features)
    b2 = b_2.astype(jnp.float32).reshape(1, features)
    w_bf16 = w.astype(jnp.bfloat16)

    block = _BLOCK_ROWS
    while rows % block:
        block //= 2
    n_steps = rows // block

    out = pl.pallas_call(
        functools.partial(_pipelined_kernel, block=block, n_steps=n_steps,
                          eps=eps),
        out_shape=jax.ShapeDtypeStruct((rows, features), x.dtype),
        in_specs=[
            pl.BlockSpec(memory_space=pl.ANY),
            pl.BlockSpec(memory_space=pltpu.VMEM),
            pl.BlockSpec(memory_space=pltpu.VMEM),
            pl.BlockSpec(memory_space=pltpu.VMEM),
        ],
        out_specs=pl.BlockSpec(memory_space=pl.ANY),
        scratch_shapes=[
            pltpu.VMEM((_NSLOTS, block, features), jnp.float32),
            pltpu.VMEM((_NSLOTS, block, features), jnp.float32),
            pltpu.SemaphoreType.DMA((_NSLOTS,)),
            pltpu.SemaphoreType.DMA((_NSLOTS, 4)),
        ],
        compiler_params=pltpu.CompilerParams(
            vmem_limit_bytes=48 * 1024 * 1024,
        ),
    )(x2, g2, b2, w_bf16)

    return out.reshape(orig_shape)
